# Initial kernel scaffold; baseline (speedup 1.0000x reference)
import jax
import jax.numpy as jnp
from jax.experimental import pallas as pl


def kernel(x, edge_index, W_a, b_a, W_b, b_b, W_c, b_c, ln_g, ln_b, Wf, bf):
    raise NotImplementedError("write your pallas kernel here")


# speedup vs baseline: 20.5422x; 20.5422x over previous
import functools

import jax
import jax.numpy as jnp
from jax import lax
from jax.experimental import pallas as pl
from jax.experimental.pallas import tpu as pltpu
from jax.experimental.pallas import tpu_sc as plsc

NC = 2
NS = 16
K = 128


def _spmm_partials(table, src_idx, dst_idx, n, nwin):
    w = table.shape[1]
    np_rows = -(-(n + 16) // (NS * K)) * NS * K
    rpt = np_rows // NS

    mesh = plsc.VectorSubcoreMesh(core_axis_name="c", subcore_axis_name="s")

    @functools.partial(
        pl.kernel,
        out_type=jax.ShapeDtypeStruct((NC, np_rows, w), jnp.float32),
        mesh=mesh,
        scratch_types=[
            pltpu.VMEM((nwin, K), jnp.int32),
            pltpu.VMEM((nwin, K), jnp.int32),
            pltpu.VMEM((K, w), jnp.float32),
            pltpu.VMEM_SHARED((np_rows, w), jnp.float32),
            pltpu.SemaphoreType.DMA,
        ],
    )
    def k(table_h, src_h, dst_h, out_h, src_v, dst_v, rows_v, acc, sem):
        c = lax.axis_index("c")
        s = lax.axis_index("s")
        pltpu.sync_copy(src_h.at[c, s], src_v)
        pltpu.sync_copy(dst_h.at[c, s], dst_v)

        def zfill(i, carry):
            for j in range(w // 16):
                rows_v[i, pl.ds(j * 16, 16)] = jnp.zeros((16,), jnp.float32)
            return carry

        lax.fori_loop(0, K, zfill, 0)
        for j in range(rpt // K):
            pltpu.sync_copy(rows_v, acc.at[pl.ds(s * rpt + j * K, K)])
        plsc.subcore_barrier()

        def body(i, carry):
            pltpu.async_copy(table_h.at[src_v.at[i]], rows_v, sem).wait()
            pltpu.sync_copy(rows_v, acc.at[dst_v.at[i]], add=True)
            return carry

        lax.fori_loop(0, nwin, body, 0)
        plsc.subcore_barrier()
        pltpu.sync_copy(acc.at[pl.ds(s * rpt, rpt)], out_h.at[c, pl.ds(s * rpt, rpt)])

    return k(table, src_idx, dst_idx)


def _degree_partials(dst_idx, n, nwin):
    ndp = -(-(n + 16) // (NS * 16)) * NS * 16
    ept = ndp // NS

    mesh = plsc.VectorSubcoreMesh(core_axis_name="c", subcore_axis_name="s")

    @functools.partial(
        pl.kernel,
        out_type=(jax.ShapeDtypeStruct((ndp,), jnp.float32),
                  jax.ShapeDtypeStruct((ndp,), jnp.float32)),
        mesh=mesh,
        scratch_types=[
            pltpu.VMEM((nwin, K), jnp.int32),
            pltpu.VMEM((K,), jnp.float32),
            pltpu.VMEM((ept,), jnp.float32),
            pltpu.VMEM_SHARED((ndp,), jnp.float32),
        ],
    )
    def k(dst_h, out0_h, out1_h, dst_v, ones_v, stage_v, acc):
        c = lax.axis_index("c")
        s = lax.axis_index("s")
        pltpu.sync_copy(dst_h.at[c, s], dst_v)
        for j in range(K // 16):
            ones_v[pl.ds(j * 16, 16)] = jnp.ones((16,), jnp.float32)
        for j in range(ept // 16):
            stage_v[pl.ds(j * 16, 16)] = jnp.zeros((16,), jnp.float32)
        pltpu.sync_copy(stage_v, acc.at[pl.ds(s * ept, ept)])
        plsc.subcore_barrier()

        def body(i, carry):
            pltpu.sync_copy(ones_v, acc.at[dst_v.at[i]], add=True)
            return carry

        lax.fori_loop(0, nwin, body, 0)
        plsc.subcore_barrier()
        pltpu.sync_copy(acc.at[pl.ds(s * ept, ept)], stage_v)

        @pl.when(c == 0)
        def _():
            pltpu.sync_copy(stage_v, out0_h.at[pl.ds(s * ept, ept)])

        @pl.when(c == 1)
        def _():
            pltpu.sync_copy(stage_v, out1_h.at[pl.ds(s * ept, ept)])

    return k(dst_idx)


def _ln(t, g, b):
    mu = jnp.mean(t, axis=-1, keepdims=True)
    var = jnp.mean(jnp.square(t - mu), axis=-1, keepdims=True)
    return g * (t - mu) * lax.rsqrt(var + 1e-5) + b


def _prep_weights(W_c, Wf3, b_c, bf2):
    d_h, d_out = W_c.shape[1], W_c.shape[2]

    def body(wc_ref, wf_ref, bc_ref, bf_ref, wcf_ref, bfold_ref):
        acc = bf_ref[...]
        for s in range(wc_ref.shape[0]):
            wcf_ref[s] = jnp.dot(wc_ref[s], wf_ref[s],
                                 preferred_element_type=jnp.float32)
            acc = acc + jnp.dot(bc_ref[s:s + 1, :], wf_ref[s],
                                preferred_element_type=jnp.float32)
        bfold_ref[...] = acc

    return pl.pallas_call(
        body,
        out_shape=(jax.ShapeDtypeStruct((3, d_h, d_out), jnp.float32),
                   jax.ShapeDtypeStruct((1, d_out), jnp.float32)),
    )(W_c, Wf3, b_c, bf2)


def _stage1(deg_t, x, rb):
    n, d = x.shape

    def body(deg_ref, x_ref, dinv_ref, xs_ref):
        dinv = lax.rsqrt(deg_ref[:, 0:1] + deg_ref[:, 1:2] + 1.0)
        dinv_ref[...] = dinv
        xs_ref[...] = x_ref[...] * dinv

    return pl.pallas_call(
        body,
        grid=(n // rb,),
        in_specs=[
            pl.BlockSpec((rb, NC), lambda i: (i, 0)),
            pl.BlockSpec((rb, d), lambda i: (i, 0)),
        ],
        out_specs=(pl.BlockSpec((rb, 1), lambda i: (i, 0)),
                   pl.BlockSpec((rb, d), lambda i: (i, 0))),
        out_shape=(jax.ShapeDtypeStruct((n, 1), jnp.float32),
                   jax.ShapeDtypeStruct((n, d), jnp.float32)),
    )(deg_t, x)


def _stage2(S0p, Xs, dinv, W_a, b_a, W_b, ln_g, ln_b, rb):
    n, d = Xs.shape
    ns = W_a.shape[0]

    def body(s0_ref, xs_ref, dinv_ref, wa_ref, ba_ref, wb_ref, g_ref, b_ref, z_ref):
        dinv = dinv_ref[...]
        p0 = dinv * (s0_ref[0] + s0_ref[1] + xs_ref[...])
        for s in range(ns):
            t = jnp.dot(p0, wa_ref[s], preferred_element_type=jnp.float32)
            t = jnp.maximum(t + ba_ref[s:s + 1, :], 0.0)
            t = _ln(t, g_ref[3 * s:3 * s + 1, :], b_ref[3 * s:3 * s + 1, :])
            z_ref[s] = jnp.dot(t, wb_ref[s], preferred_element_type=jnp.float32) * dinv

    return pl.pallas_call(
        body,
        grid=(n // rb,),
        in_specs=[
            pl.BlockSpec((NC, rb, d), lambda i: (0, i, 0)),
            pl.BlockSpec((rb, d), lambda i: (i, 0)),
            pl.BlockSpec((rb, 1), lambda i: (i, 0)),
            pl.BlockSpec(W_a.shape, lambda i: (0, 0, 0)),
            pl.BlockSpec(b_a.shape, lambda i: (0, 0)),
            pl.BlockSpec(W_b.shape, lambda i: (0, 0, 0)),
            pl.BlockSpec(ln_g.shape, lambda i: (0, 0)),
            pl.BlockSpec(ln_b.shape, lambda i: (0, 0)),
        ],
        out_specs=pl.BlockSpec((ns, rb, d), lambda i: (0, i, 0)),
        out_shape=jax.ShapeDtypeStruct((ns, n, d), jnp.float32),
    )(S0p, Xs, dinv, W_a, b_a, W_b, ln_g, ln_b)


def _stage3(S1p, Z, dinv, b_b, W_cf, ln_g, ln_b, rb):
    ns, n, d = Z.shape
    d_out = W_cf.shape[2]

    def body(s1_ref, z_ref, dinv_ref, bb_ref, wcf_ref, g_ref, b_ref, us_ref):
        dinv = dinv_ref[...]
        rows = s1_ref.shape[2]
        acc = jnp.zeros((rows, d_out), jnp.float32)
        for s in range(ns):
            p1 = dinv * (s1_ref[s, 0] + s1_ref[s, 1] + z_ref[s])
            t = jnp.maximum(p1 + bb_ref[s:s + 1, :], 0.0)
            t = _ln(t, g_ref[3 * s + 1:3 * s + 2, :], b_ref[3 * s + 1:3 * s + 2, :])
            acc = acc + jnp.dot(t, wcf_ref[s], preferred_element_type=jnp.float32)
        us_ref[...] = jnp.concatenate(
            [acc * dinv, jnp.zeros((rows, 128 - d_out), jnp.float32)], axis=-1)

    return pl.pallas_call(
        body,
        grid=(n // rb,),
        in_specs=[
            pl.BlockSpec((ns, NC, rb, d), lambda i: (0, 0, i, 0)),
            pl.BlockSpec((ns, rb, d), lambda i: (0, i, 0)),
            pl.BlockSpec((rb, 1), lambda i: (i, 0)),
            pl.BlockSpec(b_b.shape, lambda i: (0, 0)),
            pl.BlockSpec(W_cf.shape, lambda i: (0, 0, 0)),
            pl.BlockSpec(ln_g.shape, lambda i: (0, 0)),
            pl.BlockSpec(ln_b.shape, lambda i: (0, 0)),
        ],
        out_specs=pl.BlockSpec((rb, 128), lambda i: (i, 0)),
        out_shape=jax.ShapeDtypeStruct((n, 128), jnp.float32),
    )(S1p, Z, dinv, b_b, W_cf, ln_g, ln_b)


def _stage4(S2p, Us, dinv, bfold, d_out, rb):
    n, wp = Us.shape

    def body(s2_ref, us_ref, dinv_ref, bf_ref, out_ref):
        t = s2_ref[0] + s2_ref[1] + us_ref[...]
        out_ref[...] = dinv_ref[...] * t[:, :d_out] + bf_ref[...]

    return pl.pallas_call(
        body,
        grid=(n // rb,),
        in_specs=[
            pl.BlockSpec((NC, rb, wp), lambda i: (0, i, 0)),
            pl.BlockSpec((rb, wp), lambda i: (i, 0)),
            pl.BlockSpec((rb, 1), lambda i: (i, 0)),
            pl.BlockSpec((1, d_out), lambda i: (0, 0)),
        ],
        out_specs=pl.BlockSpec((rb, d_out), lambda i: (i, 0)),
        out_shape=jax.ShapeDtypeStruct((n, d_out), jnp.float32),
    )(S2p, Us, dinv, bfold)


def kernel(x, edge_index, W_a, b_a, W_b, b_b, W_c, b_c, ln_g, ln_b, Wf, bf):
    n, d_in = x.shape
    e = edge_index.shape[1]
    ns = W_a.shape[0]
    d_out = W_c.shape[2]
    rb = 1000

    ept = -(-e // (NC * NS))
    nwin = -(-ept // K)
    pad = NC * NS * nwin * K - e
    kpad = jnp.arange(pad, dtype=jnp.int32)
    src = jnp.concatenate([edge_index[0].astype(jnp.int32), (kpad * 97) % n])
    dst = jnp.concatenate([edge_index[1].astype(jnp.int32), n + (kpad % 16)])
    src_idx = src.reshape(NC, NS, nwin, K)
    dst_idx = dst.reshape(NC, NS, nwin, K)

    deg0, deg1 = _degree_partials(dst_idx, n, nwin)
    deg_t = jnp.stack([deg0, deg1], axis=1)
    dinv, Xs = _stage1(deg_t, x, rb)
    S0p = _spmm_partials(Xs, src_idx, dst_idx, n, nwin)
    Z = _stage2(S0p, Xs, dinv, W_a, b_a, W_b, ln_g, ln_b, rb)
    S1p = jnp.stack([
        _spmm_partials(Z[s], src_idx, dst_idx, n, nwin) for s in range(ns)
    ])
    W_cf, bfold = _prep_weights(W_c, Wf.reshape(ns, d_out, d_out),
                                b_c, bf.reshape(1, d_out))
    Us = _stage3(S1p, Z, dinv, b_b, W_cf, ln_g, ln_b, rb)
    S2p = _spmm_partials(Us, src_idx, dst_idx, n, nwin)
    return _stage4(S2p, Us, dinv, bfold, d_out, rb)
